# baseline (speedup 1.0000x reference)
import functools

import jax
import jax.numpy as jnp
from jax import lax
from jax.experimental import pallas as pl
from jax.experimental.pallas import tpu as pltpu
from jax.experimental.pallas import tpu_sc as plsc

BATCH = 16384
DIM = 32
PACK = 4
PROW = 128
NUM_CORES = 2
NUM_SUBCORES = 16
NW = NUM_CORES * NUM_SUBCORES
BPW = BATCH // NW
CHUNK = 128
NCHUNK = BPW // CHUNK
LANES = 16


def _gather_body(user_hbm, item_hbm, tug, tig, tum, tim,
                 oug, oig, oum, oim,
                 uidx, iidx, bug, big, bum, bim, gsem):
    wid = lax.axis_index("s") * NUM_CORES + lax.axis_index("c")
    base = wid * BPW

    pltpu.sync_copy(user_hbm.at[wid], uidx)
    pltpu.sync_copy(item_hbm.at[wid], iidx)

    def shift_step(k, carry):
        sl = pl.ds(k * LANES, LANES)
        uidx[sl] = lax.shift_right_logical(uidx[sl], 2)
        iidx[sl] = lax.shift_right_logical(iidx[sl], 2)
        return carry

    lax.fori_loop(0, BPW // LANES, shift_step, 0, unroll=8)

    for j in range(NCHUNK):
        rows = pl.ds(j * CHUNK, CHUNK)
        cs = [pltpu.async_copy(tug.at[uidx.at[rows]], bug, gsem),
              pltpu.async_copy(tig.at[iidx.at[rows]], big, gsem),
              pltpu.async_copy(tum.at[uidx.at[rows]], bum, gsem),
              pltpu.async_copy(tim.at[iidx.at[rows]], bim, gsem)]
        for c in cs:
            c.wait()
        orows = pl.ds(base + j * CHUNK, CHUNK)
        pltpu.sync_copy(bug, oug.at[orows])
        pltpu.sync_copy(big, oig.at[orows])
        pltpu.sync_copy(bum, oum.at[orows])
        pltpu.sync_copy(bim, oim.at[orows])


_sc_gather = functools.partial(
    pl.kernel,
    out_type=[jax.ShapeDtypeStruct((BATCH, PROW), jnp.float32)] * 4,
    mesh=plsc.VectorSubcoreMesh(core_axis_name="c", subcore_axis_name="s"),
    scratch_types=[
        pltpu.VMEM((BPW,), jnp.int32),
        pltpu.VMEM((BPW,), jnp.int32),
        pltpu.VMEM((CHUNK, PROW), jnp.float32),
        pltpu.VMEM((CHUNK, PROW), jnp.float32),
        pltpu.VMEM((CHUNK, PROW), jnp.float32),
        pltpu.VMEM((CHUNK, PROW), jnp.float32),
        pltpu.SemaphoreType.DMA,
    ],
)(_gather_body)


def _extract(packed, sel):
    out = packed[:, 0:DIM]
    for k in range(1, PACK):
        out = jnp.where(sel == k, packed[:, k * DIM:(k + 1) * DIM], out)
    return out


def _mlp_body(uref, iref, pug, pig, pum, pim,
              w1u, w1i, b1, w2, b2, w3, b3, wpm, wpg, bp, out):
    usel = (uref[0, 0, :] & (PACK - 1)).reshape(-1, 1)
    isel = (iref[0, 0, :] & (PACK - 1)).reshape(-1, 1)
    eug = _extract(pug[...], usel)
    eig = _extract(pig[...], isel)
    eum = _extract(pum[...], usel)
    eim = _extract(pim[...], isel)
    gmf = eug * eig
    h = (jnp.dot(eum, w1u[...], preferred_element_type=jnp.float32)
         + jnp.dot(eim, w1i[...], preferred_element_type=jnp.float32)
         + b1[...])
    h = jnp.maximum(h, 0.0)
    h = jnp.maximum(jnp.dot(h, w2[...], preferred_element_type=jnp.float32) + b2[...], 0.0)
    h = jnp.maximum(jnp.dot(h, w3[...], preferred_element_type=jnp.float32) + b3[...], 0.0)
    p = (jnp.dot(h, wpm[...], preferred_element_type=jnp.float32)
         + jnp.dot(gmf, wpg[...], preferred_element_type=jnp.float32)
         + bp[...])
    out[...] = p


def kernel(user, item, embed_user_GMF, embed_item_GMF, embed_user_MLP, embed_item_MLP,
           W1, b1, W2, b2, W3, b3, Wp, bp):
    user = user.astype(jnp.int32)
    item = item.astype(jnp.int32)
    user_w = user.reshape(NW, BPW)
    item_w = item.reshape(NW, BPW)

    tug = embed_user_GMF.reshape(-1, PROW)
    tig = embed_item_GMF.reshape(-1, PROW)
    tum = embed_user_MLP.reshape(-1, PROW)
    tim = embed_item_MLP.reshape(-1, PROW)

    pug, pig, pum, pim = _sc_gather(user_w, item_w, tug, tig, tum, tim)

    w1u = W1[:, :DIM].T
    w1i = W1[:, DIM:].T
    w2 = W2.T
    w3 = W3.T
    wpg = Wp[:, :DIM].T
    wpm = Wp[:, DIM:].T

    TB = 2048
    grid = (BATCH // TB,)
    row_spec = pl.BlockSpec((TB, PROW), lambda i: (i, 0))
    idx_spec = pl.BlockSpec((1, 1, TB), lambda i: (i, 0, 0))
    full = lambda s: pl.BlockSpec(s, lambda i: (0,) * len(s))

    pred = pl.pallas_call(
        _mlp_body,
        grid=grid,
        in_specs=[
            idx_spec, idx_spec,
            row_spec, row_spec, row_spec, row_spec,
            full(w1u.shape), full(w1i.shape), full((1, 64)),
            full(w2.shape), full((1, 32)),
            full(w3.shape), full((1, 16)),
            full(wpm.shape), full(wpg.shape), full((1, 1)),
        ],
        out_specs=pl.BlockSpec((TB, 1), lambda i: (i, 0)),
        out_shape=jax.ShapeDtypeStruct((BATCH, 1), jnp.float32),
    )(user.reshape(-1, 1, TB), item.reshape(-1, 1, TB),
      pug, pig, pum, pim,
      w1u, w1i, b1.reshape(1, 64), w2, b2.reshape(1, 32),
      w3, b3.reshape(1, 16), wpm, wpg, bp.reshape(1, 1))

    return pred.reshape(-1)
